# baseline (speedup 1.0000x reference)
import functools

import jax
import jax.numpy as jnp
from jax import lax
from jax.experimental import pallas as pl
from jax.experimental.pallas import tpu as pltpu
from jax.experimental.pallas import tpu_sc as plsc

N = 10000
E = 320000
D = 128
NG = D // 16
NC = 2
NS = 16
NW = NC * NS
EPW = E // NW
CE = 80
NCHUNK = EPW // CE
RCH = 80
NRCH = N // RCH
NRI = -(-NRCH // NS)


def _prep_body(h_ref, wt_ref, b_ref, a_ref, c_ref, p1_ref, q_ref, m_ref, w_ref):
    h = h_ref[...]
    wt = wt_ref[...]
    p1 = lax.dot_general(h, wt[:D, :], (((1,), (0,)), ((), ())),
                         precision=lax.Precision.HIGHEST,
                         preferred_element_type=jnp.float32) + b_ref[...]
    p2 = lax.dot_general(h, wt[D:, :], (((1,), (0,)), ((), ())),
                         precision=lax.Precision.HIGHEST,
                         preferred_element_type=jnp.float32)
    p1_ref[...] = p1
    q_ref[...] = jnp.concatenate([p2, h], axis=1)
    m = jnp.maximum(jnp.max(p1, axis=0, keepdims=True)
                    + jnp.max(p2, axis=0, keepdims=True), 0.0)
    m_ref[...] = jnp.broadcast_to(m, (8, D))
    a = a_ref[...]
    c = c_ref[...]
    an = (a - jnp.min(a)) / (jnp.max(a) - jnp.min(a) + 1e-8)
    cn = (c - jnp.min(c)) / (jnp.max(c) - jnp.min(c) + 1e-8)
    w_ref[...] = 0.5 * an + 0.5 * cn


def _prep(h, wt, b2, am, cn):
    return pl.pallas_call(
        _prep_body,
        out_shape=[
            jax.ShapeDtypeStruct((N, D), jnp.float32),
            jax.ShapeDtypeStruct((N, 2 * D), jnp.float32),
            jax.ShapeDtypeStruct((8, D), jnp.float32),
            jax.ShapeDtypeStruct((E // D, D), jnp.float32),
        ],
    )(h, wt, b2, am, cn)


@functools.partial(
    pl.kernel,
    mesh=plsc.VectorSubcoreMesh(core_axis_name="c", subcore_axis_name="s"),
    out_type=[
        jax.ShapeDtypeStruct((NC, N, D), jnp.float32),
        jax.ShapeDtypeStruct((NW, 8, D), jnp.float32),
    ],
    scratch_types=[
        pltpu.VMEM((CE,), jnp.int32),
        pltpu.VMEM((CE,), jnp.int32),
        pltpu.VMEM((CE + 16,), jnp.float32),
        pltpu.VMEM((CE, D), jnp.float32),
        pltpu.VMEM((CE, 2 * D), jnp.float32),
        pltpu.VMEM((CE, D), jnp.float32),
        pltpu.VMEM((D,), jnp.float32),
        pltpu.VMEM((D,), jnp.float32),
        pltpu.VMEM_SHARED((N, D), jnp.float32),
        pltpu.SemaphoreType.DMA,
    ],
)
def _sc_edge(p1_hbm, q_hbm, a0_hbm, a1_hbm, w_hbm, m_hbm,
             agg_out, z_out,
             a0_v, a1_v, w_v, r1, rq, ov, m_v, z_v, agg_sp, sem):
    cid = lax.axis_index("c")
    sid = lax.axis_index("s")
    wid = cid * NS + sid

    zero16 = jnp.zeros((16,), jnp.float32)

    def zrow(i, carry):
        for k in range(NG):
            ov[i, pl.ds(16 * k, 16)] = zero16
        return carry

    lax.fori_loop(0, RCH, zrow, 0)

    def zcopy(i, carry):
        c = sid + i * NS

        @pl.when(c < NRCH)
        def _():
            r0 = pl.multiple_of(c * RCH, 8)
            pltpu.sync_copy(ov, agg_sp.at[pl.ds(r0, RCH)])

        return carry

    lax.fori_loop(0, NRI, zcopy, 0)
    plsc.subcore_barrier()

    pltpu.sync_copy(m_hbm.at[0], m_v)
    mv = [m_v[pl.ds(16 * k, 16)] for k in range(NG)]

    base = wid * EPW

    def chunk(c, zcar):
        off = pl.multiple_of(base + c * CE, 8)
        pltpu.sync_copy(a0_hbm.at[pl.ds(off, CE)], a0_v)
        pltpu.sync_copy(a1_hbm.at[pl.ds(off, CE)], a1_v)
        pltpu.sync_copy(w_hbm.at[pl.ds(off, CE)], w_v.at[pl.ds(0, CE)])
        c1 = pltpu.async_copy(p1_hbm.at[a0_v], r1, sem)
        c2 = pltpu.async_copy(q_hbm.at[a1_v], rq, sem)
        c1.wait()
        c2.wait()

        def edge(e, zc):
            we = w_v[pl.ds(e, 16)][0]
            acc = []
            for k in range(NG):
                sl = pl.ds(16 * k, 16)
                x = r1[e, sl] + rq[e, sl]
                x = jnp.where(x >= 0.0, x, x * 0.01)
                es = jnp.exp(x * we - mv[k])
                ov[e, sl] = es * rq[e, pl.ds(D + 16 * k, 16)]
                acc.append(zc[k] + es)
            return tuple(acc)

        zcar = lax.fori_loop(0, CE, edge, zcar)
        pltpu.sync_copy(ov, agg_sp.at[a0_v], add=True)
        return zcar

    zinit = tuple(jnp.zeros((16,), jnp.float32) for _ in range(NG))
    zcar = lax.fori_loop(0, NCHUNK, chunk, zinit)

    for k in range(NG):
        z_v[pl.ds(16 * k, 16)] = zcar[k]
    pltpu.sync_copy(z_v, z_out.at[wid, 0])

    plsc.subcore_barrier()

    def ocopy(i, carry):
        c = sid + i * NS

        @pl.when(c < NRCH)
        def _():
            r0 = pl.multiple_of(c * RCH, 8)
            pltpu.sync_copy(agg_sp.at[pl.ds(r0, RCH)], ov)
            pltpu.sync_copy(ov, agg_out.at[cid, pl.ds(r0, RCH)])

        return carry

    lax.fori_loop(0, NRI, ocopy, 0)


def _fin_body(p_ref, z_ref, o_ref):
    z = jnp.sum(z_ref[:, 0, :], axis=0, keepdims=True)
    o_ref[...] = (p_ref[0] + p_ref[1]) / z


def _fin(parts, zparts):
    return pl.pallas_call(
        _fin_body,
        out_shape=jax.ShapeDtypeStruct((N, D), jnp.float32),
    )(parts, zparts)


def kernel(h, adj, amount, count, W, b):
    a0 = adj[0].astype(jnp.int32)
    a1 = adj[1].astype(jnp.int32)
    wt = W.T
    b2 = b.reshape(1, D)
    am = amount.reshape(E // D, D)
    cn = count.reshape(E // D, D)
    p1, q, m8, w2 = _prep(h, wt, b2, am, cn)
    w = w2.reshape(E)
    parts, zparts = _sc_edge(p1, q, a0, a1, w, m8)
    return _fin(parts, zparts)


# speedup vs baseline: 4.1281x; 4.1281x over previous
import functools

import jax
import jax.numpy as jnp
from jax import lax
from jax.experimental import pallas as pl
from jax.experimental.pallas import tpu as pltpu
from jax.experimental.pallas import tpu_sc as plsc

N = 10000
E = 320000
D = 128
NG = D // 16
NC = 2
NS = 16
NW = NC * NS
EPW = E // NW
CE = 40
NCHUNK = EPW // CE
NCHT = E // CE
RCH = 40
NRCH = N // RCH
NRI = -(-NRCH // NS)


def _prep_body(h_ref, wt_ref, b_ref, a_ref, c_ref, p1_ref, p2_ref, m_ref, w_ref):
    h = h_ref[...]
    wt = wt_ref[...]
    p1 = lax.dot_general(h, wt[:D, :], (((1,), (0,)), ((), ())),
                         precision=lax.Precision.HIGHEST,
                         preferred_element_type=jnp.float32) + b_ref[...]
    p2 = lax.dot_general(h, wt[D:, :], (((1,), (0,)), ((), ())),
                         precision=lax.Precision.HIGHEST,
                         preferred_element_type=jnp.float32)
    p1_ref[...] = p1
    p2_ref[...] = p2
    m = jnp.maximum(jnp.max(p1, axis=0, keepdims=True)
                    + jnp.max(p2, axis=0, keepdims=True), 0.0)
    m_ref[...] = jnp.broadcast_to(m, (8, D))
    a = a_ref[...]
    c = c_ref[...]
    an = (a - jnp.min(a)) / (jnp.max(a) - jnp.min(a) + 1e-8)
    cn = (c - jnp.min(c)) / (jnp.max(c) - jnp.min(c) + 1e-8)
    w_ref[...] = 0.5 * an + 0.5 * cn


def _prep(h, wt, b2, am, cn):
    return pl.pallas_call(
        _prep_body,
        out_shape=[
            jax.ShapeDtypeStruct((N, D), jnp.float32),
            jax.ShapeDtypeStruct((N, D), jnp.float32),
            jax.ShapeDtypeStruct((8, D), jnp.float32),
            jax.ShapeDtypeStruct((E // D, D), jnp.float32),
        ],
    )(h, wt, b2, am, cn)


@functools.partial(
    pl.kernel,
    mesh=plsc.VectorSubcoreMesh(core_axis_name="c", subcore_axis_name="s"),
    out_type=[
        jax.ShapeDtypeStruct((NC, N, D), jnp.float32),
        jax.ShapeDtypeStruct((NW, 8, D), jnp.float32),
    ],
    scratch_types=[
        pltpu.VMEM((2, 2 * CE), jnp.int32),
        pltpu.VMEM((2, CE + 16), jnp.float32),
        pltpu.VMEM((2, CE), jnp.int32),
        pltpu.VMEM((2, CE, D), jnp.float32),
        pltpu.VMEM((2, CE, D), jnp.float32),
        pltpu.VMEM((2, CE, D), jnp.float32),
        pltpu.VMEM((2, CE, D), jnp.float32),
        pltpu.VMEM((D,), jnp.float32),
        pltpu.VMEM((D,), jnp.float32),
        pltpu.VMEM_SHARED((N, D), jnp.float32),
        pltpu.SemaphoreType.DMA,
        pltpu.SemaphoreType.DMA,
        pltpu.SemaphoreType.DMA,
        pltpu.SemaphoreType.DMA,
        pltpu.SemaphoreType.DMA,
    ],
)
def _sc_edge(p1_hbm, p2_hbm, h_hbm, adjp_hbm, w_hbm, m_hbm,
             agg_out, z_out,
             idx_v, w_v, s_idx, r1, r2, rh, ov, m_v, z_v, agg_sp,
             sem_g, sem_i0, sem_i1, sem_s0, sem_s1):
    cid = lax.axis_index("c")
    sid = lax.axis_index("s")
    wid = cid * NS + sid
    sem_i = (sem_i0, sem_i1)
    sem_s = (sem_s0, sem_s1)

    zero16 = jnp.zeros((16,), jnp.float32)

    def zrow(i, carry):
        for k in range(NG):
            ov[0, i, pl.ds(16 * k, 16)] = zero16
        return carry

    lax.fori_loop(0, RCH, zrow, 0)

    def zcopy(i, carry):
        c = sid + i * NS

        @pl.when(c < NRCH)
        def _():
            r0 = pl.multiple_of(c * RCH, 8)
            pltpu.sync_copy(ov.at[0], agg_sp.at[pl.ds(r0, RCH)])

        return carry

    lax.fori_loop(0, NRI, zcopy, 0)
    plsc.subcore_barrier()

    pltpu.sync_copy(m_hbm.at[0], m_v)
    mv = [m_v[pl.ds(16 * k, 16)] for k in range(NG)]

    base = wid * EPW
    gbase = wid * NCHUNK

    def fire_idx(c, b):
        ioff = pl.multiple_of((gbase + c) * 2 * CE, 8)
        pltpu.async_copy(adjp_hbm.at[pl.ds(ioff, 2 * CE)], idx_v.at[b], sem_i[b])

    def fire_w(c, b):
        woff = pl.multiple_of(base + c * CE, 8)
        pltpu.async_copy(w_hbm.at[pl.ds(woff, CE)],
                         w_v.at[b, pl.ds(0, CE)], sem_i[b])

    def wait_idx(b):
        ioff = 0
        pltpu.make_async_copy(adjp_hbm.at[pl.ds(ioff, 2 * CE)],
                              idx_v.at[b], sem_i[b]).wait()
        pltpu.make_async_copy(w_hbm.at[pl.ds(0, CE)],
                              w_v.at[b, pl.ds(0, CE)], sem_i[b]).wait()

    def fire_gathers(b):
        pltpu.async_copy(p1_hbm.at[idx_v.at[b, pl.ds(0, CE)]], r1.at[b], sem_g)
        pltpu.async_copy(p2_hbm.at[idx_v.at[b, pl.ds(CE, CE)]], r2.at[b], sem_g)
        pltpu.async_copy(h_hbm.at[idx_v.at[b, pl.ds(CE, CE)]], rh.at[b], sem_g)

    def wait_gathers(b):
        pltpu.make_async_copy(p1_hbm.at[idx_v.at[b, pl.ds(0, CE)]],
                              r1.at[b], sem_g).wait()
        pltpu.make_async_copy(p2_hbm.at[idx_v.at[b, pl.ds(CE, CE)]],
                              r2.at[b], sem_g).wait()
        pltpu.make_async_copy(h_hbm.at[idx_v.at[b, pl.ds(CE, CE)]],
                              rh.at[b], sem_g).wait()

    def wait_scatter(b):
        pltpu.make_async_copy(ov.at[b], agg_sp.at[s_idx.at[b]], sem_s[b]).wait()

    def compute(c2, b, zc):
        def edge(e, zcc):
            we = w_v[b, pl.ds(e, 16)][0]
            acc = []
            for k in range(NG):
                sl = pl.ds(16 * k, 16)
                x = r1[b, e, sl] + r2[b, e, sl]
                x = jnp.where(x >= 0.0, x, x * 0.01)
                es = jnp.exp(x * we - mv[k])
                ov[b, e, sl] = es * rh[b, e, sl]
                acc.append(zcc[k] + es)
            return tuple(acc)

        return lax.fori_loop(0, CE, edge, zc)

    fire_idx(0, 0)
    fire_w(0, 0)
    fire_idx(1, 1)
    fire_w(1, 1)
    wait_idx(0)
    fire_gathers(0)

    def phase(c2, b, zcar):
        c = 2 * c2 + b
        wait_gathers(b)

        @pl.when(c2 > 0)
        def _():
            wait_scatter(b)

        for o in (0, 16, CE - 16):
            s_idx[b, pl.ds(o, 16)] = idx_v[b, pl.ds(o, 16)]

        @pl.when(c + 1 < NCHUNK)
        def _():
            wait_idx(1 - b)
            fire_gathers(1 - b)

        @pl.when(c + 2 < NCHUNK)
        def _():
            fire_idx(c + 2, b)

        zcar = compute(c2, b, zcar)

        @pl.when(c + 2 < NCHUNK)
        def _():
            fire_w(c + 2, b)

        pltpu.async_copy(ov.at[b], agg_sp.at[s_idx.at[b]], sem_s[b], add=True)
        return zcar

    def two_chunks(c2, zcar):
        zcar = phase(c2, 0, zcar)
        zcar = phase(c2, 1, zcar)
        return zcar

    zinit = tuple(jnp.zeros((16,), jnp.float32) for _ in range(NG))
    zcar = lax.fori_loop(0, NCHUNK // 2, two_chunks, zinit)
    wait_scatter(0)
    wait_scatter(1)

    for k in range(NG):
        z_v[pl.ds(16 * k, 16)] = zcar[k]
    pltpu.sync_copy(z_v, z_out.at[wid, 0])

    plsc.subcore_barrier()

    def ocopy(i, carry):
        c = sid + i * NS

        @pl.when(c < NRCH)
        def _():
            r0 = pl.multiple_of(c * RCH, 8)
            pltpu.sync_copy(agg_sp.at[pl.ds(r0, RCH)], ov.at[0])
            pltpu.sync_copy(ov.at[0], agg_out.at[cid, pl.ds(r0, RCH)])

        return carry

    lax.fori_loop(0, NRI, ocopy, 0)


def _fin_body(p_ref, z_ref, o_ref):
    z = jnp.sum(z_ref[:, 0, :], axis=0, keepdims=True)
    o_ref[...] = (p_ref[0] + p_ref[1]) / z


def _fin(parts, zparts):
    return pl.pallas_call(
        _fin_body,
        out_shape=jax.ShapeDtypeStruct((N, D), jnp.float32),
    )(parts, zparts)


def kernel(h, adj, amount, count, W, b):
    adjp = (adj.astype(jnp.int32).reshape(2, NCHT, CE)
            .transpose(1, 0, 2).reshape(2 * E))
    wt = W.T
    b2 = b.reshape(1, D)
    am = amount.reshape(E // D, D)
    cn = count.reshape(E // D, D)
    p1, p2, m8, w2 = _prep(h, wt, b2, am, cn)
    w = w2.reshape(E)
    parts, zparts = _sc_edge(p1, p2, h, adjp, w, m8)
    return _fin(parts, zparts)
